# baseline (speedup 1.0000x reference)
import functools

import jax
import jax.numpy as jnp
from jax import lax
from jax.experimental import pallas as pl
from jax.experimental.pallas import tpu as pltpu
from jax.experimental.pallas import tpu_sc as plsc

PLACEHOLDER = 9
B, L, D = 1024, 77, 768
TOKP = 128
LANES = 16
NC, NS = 1, 16
NW = NC * NS
RPW = B // NW
CPR = 80 // LANES
NCHUNK = RPW * CPR
MAXK = RPW * L
DH = D // 2
NCP = RPW * 2
GRP = 16


def _sc_body(tok_ref, emb_ref, text_ref, out_ref,
             tok_v, ubuf, cbuf, scr, in_sem, out_sem, g_sem, s_sem):
    w = lax.axis_index("s") * NC + lax.axis_index("c")
    b0 = w * RPW

    def in_cp(i):
        return pltpu.make_async_copy(
            emb_ref.at[b0 + i // 2, :, pl.ds((i & 1) * DH, DH)],
            cbuf.at[i & 1], in_sem.at[i & 1])

    def out_cp(i):
        return pltpu.make_async_copy(
            cbuf.at[i & 1],
            out_ref.at[b0 + i // 2, :, pl.ds((i & 1) * DH, DH)],
            out_sem.at[i & 1])

    in_cp(0).start()

    pltpu.sync_copy(tok_ref.at[pl.ds(b0, RPW)], tok_v)

    iota = lax.iota(jnp.int32, LANES)

    def scan_body(t, carry):
        k_w, row_cnt = carry
        r = t // CPR
        c = t - r * CPR
        row_cnt = jnp.where(c == 0, 0, row_cnt)
        tok16 = plsc.load_gather(
            tok_v,
            [lax.broadcast(r, (LANES,)),
             lax.broadcast(c * LANES, (LANES,)) + iota])
        mask = tok16 == PLACEHOLDER
        csum = plsc.cumsum(mask.astype(jnp.int32))
        cnt = jnp.sum(mask.astype(jnp.int32))
        rank = lax.broadcast(row_cnt - 1, (LANES,)) + csum
        dpos = lax.broadcast(r * 128 + c * LANES, (LANES,)) + iota
        packed = dpos * 128 + rank
        gslot = lax.broadcast(k_w - 1, (LANES,)) + csum
        plsc.store_scatter(ubuf, [gslot], packed, mask=mask)
        return k_w + cnt, row_cnt + cnt

    k_w, _ = lax.fori_loop(0, NCHUNK, scan_body,
                           (jnp.int32(0), jnp.int32(0)))

    def pump(i, carry):
        @pl.when(i >= 1)
        def _():
            out_cp(i - 1).wait()

        @pl.when(i + 1 < NCP)
        def _():
            in_cp(i + 1).start()

        in_cp(i).wait()
        out_cp(i).start()
        return carry

    lax.fori_loop(0, NCP, pump, jnp.int32(0))
    out_cp(NCP - 1).wait()

    def extract(j):
        v16 = plsc.load_gather(ubuf, [lax.broadcast(j, (LANES,))])
        v = jnp.max(v16)
        sl = v & 127
        rest = v >> 7
        ln = rest & 127
        return b0 + (rest >> 7), ln, sl

    def group_body(m, carry):
        cnt = jnp.minimum(k_w - m * GRP, GRP)

        def gather_one(j, carry):
            b, ln, sl = extract(m * GRP + j)
            pltpu.make_async_copy(text_ref.at[b, sl], scr.at[j],
                                  g_sem).start()
            return carry

        def gdrain_one(j, carry):
            pltpu.make_async_copy(text_ref.at[b0, 0], scr.at[0],
                                  g_sem).wait()
            return carry

        def scatter_one(j, carry):
            b, ln, sl = extract(m * GRP + j)
            pltpu.make_async_copy(scr.at[j], out_ref.at[b, ln],
                                  s_sem).start()
            return carry

        def sdrain_one(j, carry):
            pltpu.make_async_copy(scr.at[0], out_ref.at[b0, 0],
                                  s_sem).wait()
            return carry

        lax.fori_loop(0, cnt, gather_one, jnp.int32(0))
        lax.fori_loop(0, cnt, gdrain_one, jnp.int32(0))
        lax.fori_loop(0, cnt, scatter_one, jnp.int32(0))
        lax.fori_loop(0, cnt, sdrain_one, jnp.int32(0))
        return carry

    lax.fori_loop(0, (k_w + GRP - 1) // GRP, group_body, jnp.int32(0))


@functools.partial(
    pl.kernel,
    out_type=jax.ShapeDtypeStruct((B, L, D), jnp.float32),
    mesh=plsc.VectorSubcoreMesh(core_axis_name="c", subcore_axis_name="s",
                                num_cores=NC, num_subcores=NS),
    compiler_params=pltpu.CompilerParams(needs_layout_passes=False,
                                         use_tc_tiling_on_sc=True),
    scratch_types=[
        pltpu.VMEM((RPW, TOKP), jnp.int32),
        pltpu.VMEM((MAXK,), jnp.int32),
        pltpu.VMEM((2, L, DH), jnp.float32),
        pltpu.VMEM((GRP, D), jnp.float32),
        pltpu.SemaphoreType.DMA((2,)),
        pltpu.SemaphoreType.DMA((2,)),
        pltpu.SemaphoreType.DMA,
        pltpu.SemaphoreType.DMA,
    ],
)
def _sc_scatter_overwrite(tok_ref, emb_ref, text_ref, out_ref,
                          tok_v, ubuf, cbuf, scr,
                          in_sem, out_sem, g_sem, s_sem):
    _sc_body(tok_ref, emb_ref, text_ref, out_ref,
             tok_v, ubuf, cbuf, scr, in_sem, out_sem, g_sem, s_sem)


@jax.jit
def _run(tok_p, embedded_text, text_embs):
    return _sc_scatter_overwrite(tok_p, embedded_text, text_embs)


def kernel(tokenized_text, embedded_text, text_embs):
    tok_p = jnp.pad(tokenized_text, ((0, 0), (0, TOKP - L)),
                    constant_values=-1)
    return _run(tok_p, embedded_text, text_embs)


# speedup vs baseline: 1.0763x; 1.0763x over previous
import functools

import jax
import jax.numpy as jnp
from jax import lax
from jax.experimental import pallas as pl
from jax.experimental.pallas import tpu as pltpu
from jax.experimental.pallas import tpu_sc as plsc

PLACEHOLDER = 9
B, L, D = 1024, 77, 768
TOKP = 128
LANES = 16
NC, NS = 2, 16
NW = NC * NS
RPW = B // NW
CPR = 80 // LANES
NCHUNK = RPW * CPR
MAXK = RPW * L
DH = D // 3
NCP = RPW * 3
RING = 4
GRP = 16


def _sc_body(tok_ref, emb_ref, text_ref, out_ref,
             tok_v, ubuf, cbuf, scr, in_sem, out_sem, g_sem, s_sem):
    w = lax.axis_index("s") * NC + lax.axis_index("c")
    b0 = w * RPW

    def in_cp(i):
        return pltpu.make_async_copy(
            emb_ref.at[b0 + i // 3, :, pl.ds((i % 3) * DH, DH)],
            cbuf.at[i % RING], in_sem.at[i % RING])

    def out_cp(i):
        return pltpu.make_async_copy(
            cbuf.at[i % RING],
            out_ref.at[b0 + i // 3, :, pl.ds((i % 3) * DH, DH)],
            out_sem.at[i % RING])

    for i in range(RING - 1):
        in_cp(i).start()

    pltpu.sync_copy(tok_ref.at[pl.ds(b0, RPW)], tok_v)

    iota = lax.iota(jnp.int32, LANES)

    def scan_body(t, carry):
        k_w, row_cnt = carry
        r = t // CPR
        c = t - r * CPR
        row_cnt = jnp.where(c == 0, 0, row_cnt)
        tok16 = plsc.load_gather(
            tok_v,
            [lax.broadcast(r, (LANES,)),
             lax.broadcast(c * LANES, (LANES,)) + iota])
        mask = tok16 == PLACEHOLDER
        csum = plsc.cumsum(mask.astype(jnp.int32))
        cnt = jnp.sum(mask.astype(jnp.int32))
        rank = lax.broadcast(row_cnt - 1, (LANES,)) + csum
        dpos = lax.broadcast(r * 128 + c * LANES, (LANES,)) + iota
        packed = dpos * 128 + rank
        gslot = lax.broadcast(k_w - 1, (LANES,)) + csum
        plsc.store_scatter(ubuf, [gslot], packed, mask=mask)
        return k_w + cnt, row_cnt + cnt

    k_w, _ = lax.fori_loop(0, NCHUNK, scan_body,
                           (jnp.int32(0), jnp.int32(0)))

    def pump(i, carry):
        @pl.when(i >= 1)
        def _():
            out_cp(i - 1).wait()

        @pl.when(i + RING - 1 < NCP)
        def _():
            in_cp(i + RING - 1).start()

        in_cp(i).wait()
        out_cp(i).start()
        return carry

    lax.fori_loop(0, NCP, pump, jnp.int32(0))
    out_cp(NCP - 1).wait()

    def extract(j):
        v16 = plsc.load_gather(ubuf, [lax.broadcast(j, (LANES,))])
        v = jnp.max(v16)
        sl = v & 127
        rest = v >> 7
        ln = rest & 127
        return b0 + (rest >> 7), ln, sl

    def group_body(m, carry):
        cnt = jnp.minimum(k_w - m * GRP, GRP)

        def gather_one(j, carry):
            b, ln, sl = extract(m * GRP + j)
            pltpu.make_async_copy(text_ref.at[b, sl], scr.at[j],
                                  g_sem).start()
            return carry

        def gdrain_one(j, carry):
            pltpu.make_async_copy(text_ref.at[b0, 0], scr.at[0],
                                  g_sem).wait()
            return carry

        def scatter_one(j, carry):
            b, ln, sl = extract(m * GRP + j)
            pltpu.make_async_copy(scr.at[j], out_ref.at[b, ln],
                                  s_sem).start()
            return carry

        def sdrain_one(j, carry):
            pltpu.make_async_copy(scr.at[0], out_ref.at[b0, 0],
                                  s_sem).wait()
            return carry

        lax.fori_loop(0, cnt, gather_one, jnp.int32(0))
        lax.fori_loop(0, cnt, gdrain_one, jnp.int32(0))
        lax.fori_loop(0, cnt, scatter_one, jnp.int32(0))
        lax.fori_loop(0, cnt, sdrain_one, jnp.int32(0))
        return carry

    lax.fori_loop(0, (k_w + GRP - 1) // GRP, group_body, jnp.int32(0))


@functools.partial(
    pl.kernel,
    out_type=jax.ShapeDtypeStruct((B, L, D), jnp.float32),
    mesh=plsc.VectorSubcoreMesh(core_axis_name="c", subcore_axis_name="s",
                                num_cores=NC, num_subcores=NS),
    compiler_params=pltpu.CompilerParams(needs_layout_passes=False,
                                         use_tc_tiling_on_sc=True),
    scratch_types=[
        pltpu.VMEM((RPW, TOKP), jnp.int32),
        pltpu.VMEM((MAXK,), jnp.int32),
        pltpu.VMEM((RING, L, DH), jnp.float32),
        pltpu.VMEM((GRP, D), jnp.float32),
        pltpu.SemaphoreType.DMA((RING,)),
        pltpu.SemaphoreType.DMA((RING,)),
        pltpu.SemaphoreType.DMA,
        pltpu.SemaphoreType.DMA,
    ],
)
def _sc_scatter_overwrite(tok_ref, emb_ref, text_ref, out_ref,
                          tok_v, ubuf, cbuf, scr,
                          in_sem, out_sem, g_sem, s_sem):
    _sc_body(tok_ref, emb_ref, text_ref, out_ref,
             tok_v, ubuf, cbuf, scr, in_sem, out_sem, g_sem, s_sem)


@jax.jit
def _run(tok_p, embedded_text, text_embs):
    return _sc_scatter_overwrite(tok_p, embedded_text, text_embs)


def kernel(tokenized_text, embedded_text, text_embs):
    tok_p = jnp.pad(tokenized_text, ((0, 0), (0, TOKP - L)),
                    constant_values=-1)
    return _run(tok_p, embedded_text, text_embs)
